# baseline (speedup 1.0000x reference)
import functools

import jax
import jax.numpy as jnp
from jax import lax
from jax.experimental import pallas as pl
from jax.experimental.pallas import tpu as pltpu
from jax.experimental.pallas import tpu_sc as plsc

N = 10000
E = 320000
NC, NS = 2, 16
EPT = 10240
K = 128
CPT = EPT // K
KA = 128
CPTA = EPT // KA
SG = 16
NSUP = CPTA // SG
NBUF = 2
NGRP = SG // NBUF
ACC_R = 10112
RPT = ACC_R // NS
DC = 128
TRASH = N

_MESH_KW = dict(core_axis_name="c", subcore_axis_name="s",
                num_cores=NC, num_subcores=NS)


def _make_agg():
    mesh = plsc.VectorSubcoreMesh(**_MESH_KW)

    @functools.partial(
        pl.kernel,
        out_type=jax.ShapeDtypeStruct((2 * ACC_R, DC), jnp.float32),
        mesh=mesh,
        scratch_types=(
            [pltpu.VMEM((SG, KA), jnp.int32),
             pltpu.VMEM((SG, KA), jnp.int32)]
            + [pltpu.VMEM((KA, DC), jnp.float32) for _ in range(NBUF)]
            + [pltpu.VMEM_SHARED((ACC_R, DC), jnp.float32)]
            + [pltpu.SemaphoreType.DMA for _ in range(2 * NBUF)]
        ),
    )
    def agg(y_hbm, src_hbm, dst_hbm, out_hbm, srcv, dstv, *rest):
        bufs = rest[:NBUF]
        acc = rest[NBUF]
        semg = rest[NBUF + 1:NBUF + 1 + NBUF]
        sems = rest[NBUF + 1 + NBUF:]
        c = lax.axis_index("c")
        s = lax.axis_index("s")
        w = c * NS + s
        pltpu.sync_copy(y_hbm.at[pl.ds(c * ACC_R + s * RPT, RPT)],
                        acc.at[pl.ds(s * RPT, RPT)])
        plsc.subcore_barrier()

        dummy = y_hbm.at[pl.ds(0, KA)]
        for g in range(NSUP):
            pltpu.sync_copy(src_hbm.at[w, pl.ds(g * SG, SG)], srcv)
            pltpu.sync_copy(dst_hbm.at[w, pl.ds(g * SG, SG)], dstv)
            for q in range(NBUF):
                pltpu.async_copy(y_hbm.at[srcv.at[q]], bufs[q], semg[q])

            def group(b, carry):
                for q in range(NBUF):
                    pltpu.make_async_copy(dummy, bufs[q], semg[q]).wait()
                    pltpu.async_copy(bufs[q], acc.at[dstv.at[NBUF * b + q]],
                                     sems[q], add=True)

                @pl.when(b < NGRP - 1)
                def _():
                    for q in range(NBUF):
                        pltpu.make_async_copy(dummy, bufs[q], sems[q]).wait()
                        pltpu.async_copy(
                            y_hbm.at[srcv.at[NBUF * (b + 1) + q]],
                            bufs[q], semg[q])
                return carry

            lax.fori_loop(0, NGRP, group, 0)
            for q in range(NBUF):
                pltpu.make_async_copy(dummy, bufs[q], sems[q]).wait()

        plsc.subcore_barrier()
        pltpu.sync_copy(acc.at[pl.ds(s * RPT, RPT)],
                        out_hbm.at[pl.ds(c * ACC_R + s * RPT, RPT)])

    return agg


def _make_deg():
    mesh = plsc.VectorSubcoreMesh(**_MESH_KW)

    @functools.partial(
        pl.kernel,
        out_type=jax.ShapeDtypeStruct((2 * ACC_R, 16), jnp.float32),
        mesh=mesh,
        scratch_types=[
            pltpu.VMEM((CPT, K), jnp.int32),
            pltpu.VMEM((K, 16), jnp.float32),
            pltpu.VMEM_SHARED((ACC_R, 16), jnp.float32),
        ],
    )
    def deg(dst_hbm, init_hbm, out_hbm, dstv, ones_v, acc):
        c = lax.axis_index("c")
        s = lax.axis_index("s")
        pltpu.sync_copy(dst_hbm.at[c * NS + s], dstv)
        pltpu.sync_copy(init_hbm.at[pl.ds(0, K)], ones_v)
        pltpu.sync_copy(init_hbm.at[pl.ds(c * ACC_R + s * RPT, RPT)],
                        acc.at[pl.ds(s * RPT, RPT)])
        plsc.subcore_barrier()

        def body(j, carry):
            pltpu.sync_copy(ones_v, acc.at[dstv.at[j]], add=True)
            return carry

        lax.fori_loop(0, CPT, body, 0)
        plsc.subcore_barrier()
        pltpu.sync_copy(acc.at[pl.ds(s * RPT, RPT)],
                        out_hbm.at[pl.ds(c * ACC_R + s * RPT, RPT)])

    return deg


_DEG = _make_deg()
_AGG = _make_agg()


def _mm(x, w, dis, b, *, pre_scale=False, pre_bias=False, pre_relu=False,
        post_bias=False, post_relu=False, post_scale=False, bm=1000):
    n, kdim = x.shape
    dout = w.shape[1]
    b2d = b.reshape(1, -1)
    dis2d = dis.reshape(-1, 1)

    def body(x_ref, w_ref, d_ref, b_ref, o_ref):
        a = x_ref[...]
        d = d_ref[...]
        if pre_scale:
            a = a * d
        if pre_bias:
            a = a + b_ref[...]
        if pre_relu:
            a = jnp.maximum(a, 0.0)
        acc = jnp.dot(a, w_ref[...], preferred_element_type=jnp.float32)
        if post_bias:
            acc = acc + b_ref[...]
        if post_relu:
            acc = jnp.maximum(acc, 0.0)
        if post_scale:
            acc = acc * d
        o_ref[...] = acc

    return pl.pallas_call(
        body,
        grid=(n // bm,),
        in_specs=[
            pl.BlockSpec((bm, kdim), lambda i: (i, 0)),
            pl.BlockSpec((kdim, dout), lambda i: (0, 0)),
            pl.BlockSpec((bm, 1), lambda i: (i, 0)),
            pl.BlockSpec((1, b2d.shape[1]), lambda i: (0, 0)),
        ],
        out_specs=pl.BlockSpec((bm, dout), lambda i: (i, 0)),
        out_shape=jax.ShapeDtypeStruct((n, dout), jnp.float32),
    )(x, w, dis2d, b2d)


def kernel(x, edge_index, W1, b1, W2, b2, W3, b3):
    ei = edge_index.astype(jnp.int32)
    src, dst = ei[0], ei[1]
    nw = NC * NS
    ept_real = E // nw
    pad = EPT - ept_real
    src_pad = jnp.concatenate(
        [src.reshape(nw, ept_real), jnp.zeros((nw, pad), jnp.int32)], axis=1)
    dst_pad = jnp.concatenate(
        [dst.reshape(nw, ept_real),
         jnp.full((nw, pad), TRASH, jnp.int32)], axis=1)
    src3d = src_pad.reshape(nw, CPTA, KA)
    dst3d = dst_pad.reshape(nw, CPTA, KA)
    dst3d_deg = dst_pad.reshape(nw, CPT, K)
    init_deg = jnp.zeros((2 * ACC_R, 16), jnp.float32).at[:N].set(1.0)

    dego = _DEG(dst3d_deg, init_deg)
    deg = dego[:N, 0] + dego[ACC_R:ACC_R + N, 0]
    dis = lax.rsqrt(deg)

    def agg(y):
        ys = jnp.zeros((2 * ACC_R, DC), jnp.float32).at[:N].set(y)
        o = _AGG(ys, src3d, dst3d)
        return o[:N] + o[ACC_R:ACC_R + N]

    y0 = x * dis[:, None]
    z0 = agg(y0)
    h1 = _mm(z0, W1, dis, b1, pre_scale=True, post_bias=True, post_relu=True)
    y2 = _mm(h1, W2, dis, jnp.zeros((W2.shape[1],), jnp.float32),
             post_scale=True)
    z2 = jnp.concatenate([agg(y2[:, :DC]), agg(y2[:, DC:])], axis=1)
    y3 = _mm(z2, W3, dis, b2, pre_scale=True, pre_bias=True, pre_relu=True,
             post_scale=True)
    z3 = agg(y3)
    return z3 * dis[:, None] + b3


# speedup vs baseline: 1.0702x; 1.0702x over previous
import functools

import jax
import jax.numpy as jnp
from jax import lax
from jax.experimental import pallas as pl
from jax.experimental.pallas import tpu as pltpu
from jax.experimental.pallas import tpu_sc as plsc

N = 10000
E = 320000
NC, NS = 2, 16
EPT = 10240
K = 128
CPT = EPT // K
KA = 128
CPTA = EPT // KA
SG = 16
NSUP = CPTA // SG
NBUF = 2
NGRP = SG // NBUF
ACC_R = 10112
RPT = ACC_R // NS
DC = 128
TRASH = N

_MESH_KW = dict(core_axis_name="c", subcore_axis_name="s",
                num_cores=NC, num_subcores=NS)


def _make_agg():
    mesh = plsc.VectorSubcoreMesh(**_MESH_KW)

    @functools.partial(
        pl.kernel,
        out_type=jax.ShapeDtypeStruct((2 * ACC_R, DC), jnp.float32),
        mesh=mesh,
        scratch_types=[
            pltpu.VMEM((SG, KA), jnp.int32),
            pltpu.VMEM((SG, KA), jnp.int32),
            pltpu.VMEM((KA, DC), jnp.float32),
            pltpu.VMEM((KA, DC), jnp.float32),
            pltpu.VMEM_SHARED((ACC_R, DC), jnp.float32),
            pltpu.SemaphoreType.DMA,
            pltpu.SemaphoreType.DMA,
        ],
    )
    def agg(y_hbm, src_hbm, dst_hbm, out_hbm,
            srcv, dstv, buf0, buf1, acc, sem0, sem1):
        c = lax.axis_index("c")
        s = lax.axis_index("s")
        w = c * NS + s
        pltpu.sync_copy(y_hbm.at[pl.ds(c * ACC_R + s * RPT, RPT)],
                        acc.at[pl.ds(s * RPT, RPT)])
        plsc.subcore_barrier()

        dummy = y_hbm.at[pl.ds(0, KA)]
        for g in range(NSUP):
            pltpu.sync_copy(src_hbm.at[w, pl.ds(g * SG, SG)], srcv)
            pltpu.sync_copy(dst_hbm.at[w, pl.ds(g * SG, SG)], dstv)
            pltpu.async_copy(y_hbm.at[srcv.at[0]], buf0, sem0)

            def pair(j, carry):
                pltpu.async_copy(y_hbm.at[srcv.at[2 * j + 1]], buf1, sem1)
                pltpu.make_async_copy(dummy, buf0, sem0).wait()
                pltpu.sync_copy(buf0, acc.at[dstv.at[2 * j]], add=True)

                @pl.when(j < SG // 2 - 1)
                def _():
                    pltpu.async_copy(y_hbm.at[srcv.at[2 * j + 2]], buf0, sem0)

                pltpu.make_async_copy(dummy, buf1, sem1).wait()
                pltpu.sync_copy(buf1, acc.at[dstv.at[2 * j + 1]], add=True)
                return carry

            lax.fori_loop(0, SG // 2, pair, 0)

        plsc.subcore_barrier()
        pltpu.sync_copy(acc.at[pl.ds(s * RPT, RPT)],
                        out_hbm.at[pl.ds(c * ACC_R + s * RPT, RPT)])

    return agg


def _make_deg():
    mesh = plsc.VectorSubcoreMesh(**_MESH_KW)

    @functools.partial(
        pl.kernel,
        out_type=jax.ShapeDtypeStruct((2 * ACC_R, 16), jnp.float32),
        mesh=mesh,
        scratch_types=[
            pltpu.VMEM((CPT, K), jnp.int32),
            pltpu.VMEM((K, 16), jnp.float32),
            pltpu.VMEM_SHARED((ACC_R, 16), jnp.float32),
        ],
    )
    def deg(dst_hbm, init_hbm, out_hbm, dstv, ones_v, acc):
        c = lax.axis_index("c")
        s = lax.axis_index("s")
        pltpu.sync_copy(dst_hbm.at[c * NS + s], dstv)
        pltpu.sync_copy(init_hbm.at[pl.ds(0, K)], ones_v)
        pltpu.sync_copy(init_hbm.at[pl.ds(c * ACC_R + s * RPT, RPT)],
                        acc.at[pl.ds(s * RPT, RPT)])
        plsc.subcore_barrier()

        def body(j, carry):
            pltpu.sync_copy(ones_v, acc.at[dstv.at[j]], add=True)
            return carry

        lax.fori_loop(0, CPT, body, 0)
        plsc.subcore_barrier()
        pltpu.sync_copy(acc.at[pl.ds(s * RPT, RPT)],
                        out_hbm.at[pl.ds(c * ACC_R + s * RPT, RPT)])

    return deg


_DEG = _make_deg()
_AGG = _make_agg()


def _mm(x, w, dis, b, *, pre_scale=False, pre_bias=False, pre_relu=False,
        post_bias=False, post_relu=False, post_scale=False, bm=1000):
    n, kdim = x.shape
    dout = w.shape[1]
    b2d = b.reshape(1, -1)
    dis2d = dis.reshape(-1, 1)

    def body(x_ref, w_ref, d_ref, b_ref, o_ref):
        a = x_ref[...]
        d = d_ref[...]
        if pre_scale:
            a = a * d
        if pre_bias:
            a = a + b_ref[...]
        if pre_relu:
            a = jnp.maximum(a, 0.0)
        acc = jnp.dot(a, w_ref[...], preferred_element_type=jnp.float32)
        if post_bias:
            acc = acc + b_ref[...]
        if post_relu:
            acc = jnp.maximum(acc, 0.0)
        if post_scale:
            acc = acc * d
        o_ref[...] = acc

    return pl.pallas_call(
        body,
        grid=(n // bm,),
        in_specs=[
            pl.BlockSpec((bm, kdim), lambda i: (i, 0)),
            pl.BlockSpec((kdim, dout), lambda i: (0, 0)),
            pl.BlockSpec((bm, 1), lambda i: (i, 0)),
            pl.BlockSpec((1, b2d.shape[1]), lambda i: (0, 0)),
        ],
        out_specs=pl.BlockSpec((bm, dout), lambda i: (i, 0)),
        out_shape=jax.ShapeDtypeStruct((n, dout), jnp.float32),
    )(x, w, dis2d, b2d)


def kernel(x, edge_index, W1, b1, W2, b2, W3, b3):
    ei = edge_index.astype(jnp.int32)
    src, dst = ei[0], ei[1]
    nw = NC * NS
    ept_real = E // nw
    pad = EPT - ept_real
    src_pad = jnp.concatenate(
        [src.reshape(nw, ept_real), jnp.zeros((nw, pad), jnp.int32)], axis=1)
    dst_pad = jnp.concatenate(
        [dst.reshape(nw, ept_real),
         jnp.full((nw, pad), TRASH, jnp.int32)], axis=1)
    src3d = src_pad.reshape(nw, CPTA, KA)
    dst3d = dst_pad.reshape(nw, CPTA, KA)
    dst3d_deg = dst_pad.reshape(nw, CPT, K)
    init_deg = jnp.zeros((2 * ACC_R, 16), jnp.float32).at[:N].set(1.0)

    dego = _DEG(dst3d_deg, init_deg)
    deg = dego[:N, 0] + dego[ACC_R:ACC_R + N, 0]
    dis = lax.rsqrt(deg)

    def agg(y):
        ys = jnp.zeros((2 * ACC_R, DC), jnp.float32).at[:N].set(y)
        o = _AGG(ys, src3d, dst3d)
        return o[:N] + o[ACC_R:ACC_R + N]

    y0 = x * dis[:, None]
    z0 = agg(y0)
    h1 = _mm(z0, W1, dis, b1, pre_scale=True, post_bias=True, post_relu=True)
    y2 = _mm(h1, W2, dis, jnp.zeros((W2.shape[1],), jnp.float32),
             post_scale=True)
    z2 = jnp.concatenate([agg(y2[:, :DC]), agg(y2[:, DC:])], axis=1)
    y3 = _mm(z2, W3, dis, b2, pre_scale=True, pre_bias=True, pre_relu=True,
             post_scale=True)
    z3 = agg(y3)
    return z3 * dis[:, None] + b3


# speedup vs baseline: 1.0707x; 1.0005x over previous
import functools

import jax
import jax.numpy as jnp
from jax import lax
from jax.experimental import pallas as pl
from jax.experimental.pallas import tpu as pltpu
from jax.experimental.pallas import tpu_sc as plsc

N = 10000
E = 320000
NC, NS = 2, 16
EPT = 10240
K = 128
CPT = EPT // K
KA = 128
CPTA = EPT // KA
SG = 16
NSUP = CPTA // SG
ACC_R = 10112
RPT = ACC_R // NS
DC = 128
TRASH = N

_MESH_KW = dict(core_axis_name="c", subcore_axis_name="s",
                num_cores=NC, num_subcores=NS)


def _make_agg():
    mesh = plsc.VectorSubcoreMesh(**_MESH_KW)

    @functools.partial(
        pl.kernel,
        out_type=jax.ShapeDtypeStruct((2 * ACC_R, DC), jnp.float32),
        mesh=mesh,
        scratch_types=[
            pltpu.VMEM((SG, KA), jnp.int32),
            pltpu.VMEM((SG, KA), jnp.int32),
            pltpu.VMEM((KA, DC), jnp.float32),
            pltpu.VMEM((KA, DC), jnp.float32),
            pltpu.VMEM_SHARED((ACC_R, DC), jnp.float32),
            pltpu.SemaphoreType.DMA,
            pltpu.SemaphoreType.DMA,
        ],
    )
    def agg(y_hbm, src_hbm, dst_hbm, out_hbm,
            srcv, dstv, buf0, buf1, acc, sem0, sem1):
        c = lax.axis_index("c")
        s = lax.axis_index("s")
        w = c * NS + s
        pltpu.sync_copy(y_hbm.at[pl.ds(c * ACC_R + s * RPT, RPT)],
                        acc.at[pl.ds(s * RPT, RPT)])
        plsc.subcore_barrier()

        dummy = y_hbm.at[pl.ds(0, KA)]
        for g in range(NSUP):
            pltpu.sync_copy(src_hbm.at[w, pl.ds(g * SG, SG)], srcv)
            pltpu.sync_copy(dst_hbm.at[w, pl.ds(g * SG, SG)], dstv)
            pltpu.async_copy(y_hbm.at[srcv.at[0]], buf0, sem0)

            def pair(j, carry):
                pltpu.async_copy(y_hbm.at[srcv.at[2 * j + 1]], buf1, sem1)
                pltpu.make_async_copy(dummy, buf0, sem0).wait()
                pltpu.sync_copy(buf0, acc.at[dstv.at[2 * j]], add=True)

                @pl.when(j < SG // 2 - 1)
                def _():
                    pltpu.async_copy(y_hbm.at[srcv.at[2 * j + 2]], buf0, sem0)

                pltpu.make_async_copy(dummy, buf1, sem1).wait()
                pltpu.sync_copy(buf1, acc.at[dstv.at[2 * j + 1]], add=True)
                return carry

            lax.fori_loop(0, SG // 2, pair, 0)

        plsc.subcore_barrier()
        pltpu.sync_copy(acc.at[pl.ds(s * RPT, RPT)],
                        out_hbm.at[pl.ds(c * ACC_R + s * RPT, RPT)])

    return agg


def _make_deg():
    mesh = plsc.VectorSubcoreMesh(**_MESH_KW)

    @functools.partial(
        pl.kernel,
        out_type=jax.ShapeDtypeStruct((2 * ACC_R, 16), jnp.float32),
        mesh=mesh,
        scratch_types=[
            pltpu.VMEM((CPT, K), jnp.int32),
            pltpu.VMEM((K, 16), jnp.float32),
            pltpu.VMEM_SHARED((ACC_R, 16), jnp.float32),
        ],
    )
    def deg(dst_hbm, init_hbm, out_hbm, dstv, ones_v, acc):
        c = lax.axis_index("c")
        s = lax.axis_index("s")
        pltpu.sync_copy(dst_hbm.at[c * NS + s], dstv)
        pltpu.sync_copy(init_hbm.at[pl.ds(0, K)], ones_v)
        pltpu.sync_copy(init_hbm.at[pl.ds(c * ACC_R + s * RPT, RPT)],
                        acc.at[pl.ds(s * RPT, RPT)])
        plsc.subcore_barrier()

        def body(j, carry):
            pltpu.sync_copy(ones_v, acc.at[dstv.at[j]], add=True)
            return carry

        lax.fori_loop(0, CPT, body, 0)
        plsc.subcore_barrier()
        pltpu.sync_copy(acc.at[pl.ds(s * RPT, RPT)],
                        out_hbm.at[pl.ds(c * ACC_R + s * RPT, RPT)])

    return deg


_DEG = _make_deg()
_AGG = _make_agg()


def _mm(x, w, dis, b, *, pre_scale=False, pre_bias=False, pre_relu=False,
        post_bias=False, post_relu=False, post_scale=False, bm=1000):
    n, kdim = x.shape
    dout = w.shape[1]
    b2d = b.reshape(1, -1)
    dis2d = dis.reshape(-1, 1)

    def body(x_ref, w_ref, d_ref, b_ref, o_ref):
        a = x_ref[...]
        d = d_ref[...]
        if pre_scale:
            a = a * d
        if pre_bias:
            a = a + b_ref[...]
        if pre_relu:
            a = jnp.maximum(a, 0.0)
        acc = jnp.dot(a, w_ref[...], preferred_element_type=jnp.float32)
        if post_bias:
            acc = acc + b_ref[...]
        if post_relu:
            acc = jnp.maximum(acc, 0.0)
        if post_scale:
            acc = acc * d
        o_ref[...] = acc

    return pl.pallas_call(
        body,
        grid=(n // bm,),
        in_specs=[
            pl.BlockSpec((bm, kdim), lambda i: (i, 0)),
            pl.BlockSpec((kdim, dout), lambda i: (0, 0)),
            pl.BlockSpec((bm, 1), lambda i: (i, 0)),
            pl.BlockSpec((1, b2d.shape[1]), lambda i: (0, 0)),
        ],
        out_specs=pl.BlockSpec((bm, dout), lambda i: (i, 0)),
        out_shape=jax.ShapeDtypeStruct((n, dout), jnp.float32),
    )(x, w, dis2d, b2d)


def kernel(x, edge_index, W1, b1, W2, b2, W3, b3):
    ei = edge_index.astype(jnp.int32)
    src, dst = ei[0], ei[1]
    nw = NC * NS
    ept_real = E // nw
    pad = EPT - ept_real
    src_pad = jnp.concatenate(
        [src.reshape(nw, ept_real), jnp.zeros((nw, pad), jnp.int32)], axis=1)
    dst_pad = jnp.concatenate(
        [dst.reshape(nw, ept_real),
         jnp.full((nw, pad), TRASH, jnp.int32)], axis=1)
    src3d = src_pad.reshape(nw, CPTA, KA)
    dst3d = dst_pad.reshape(nw, CPTA, KA)
    dst3d_deg = dst_pad.reshape(nw, CPT, K)
    init_deg = jnp.zeros((2 * ACC_R, 16), jnp.float32).at[:N].set(1.0)

    dego = _DEG(dst3d_deg, init_deg)
    deg = dego[:N, 0] + dego[ACC_R:ACC_R + N, 0]
    dis = lax.rsqrt(deg)

    def agg(y):
        ys = jnp.zeros((2 * ACC_R, DC), jnp.float32).at[:N].set(y)
        o = _AGG(ys, src3d, dst3d)
        return o[:N] + o[ACC_R:ACC_R + N]

    y0 = x * dis[:, None]
    z0 = agg(y0)
    h1 = _mm(z0, W1, dis, b1, pre_scale=True, post_bias=True, post_relu=True)
    y2 = _mm(h1, W2, dis, jnp.zeros((W2.shape[1],), jnp.float32),
             post_scale=True)
    z2 = jnp.concatenate([agg(y2[:, :DC]), agg(y2[:, DC:])], axis=1)
    y3 = _mm(z2, W3, dis, b2, pre_scale=True, pre_bias=True, pre_relu=True,
             post_scale=True)
    z3 = agg(y3)
    return z3 * dis[:, None] + b3
